# baseline (speedup 1.0000x reference)
import functools

import jax
import jax.numpy as jnp
from jax import lax
from jax.experimental import pallas as pl
from jax.experimental.pallas import tpu as pltpu
from jax.experimental.pallas import tpu_sc as plsc

N = 10000
D = 128
E = 320000

NC = 2
NS = 16
NW = NC * NS
EW = E // NW
CH = 80
NCH = EW // CH
NP = 10240
RP = NP // NS

_MESH = plsc.VectorSubcoreMesh(core_axis_name="c", subcore_axis_name="s")


def _wid():
    return lax.axis_index("s") * NC + lax.axis_index("c")


@functools.partial(
    pl.kernel,
    mesh=_MESH,
    out_type=jax.ShapeDtypeStruct((NW, NP // 16, 16), jnp.float32),
    compiler_params=pltpu.CompilerParams(needs_layout_passes=False),
    scratch_types=[
        pltpu.VMEM((NCH, CH), jnp.int32),
        pltpu.VMEM((NP // 16, 16), jnp.float32),
    ],
)
def _sc_degree_local(col_hbm, out_hbm, colv, degl):
    wid = _wid()
    zero16 = jnp.zeros((16,), jnp.float32)
    one16 = jnp.ones((16,), jnp.float32)

    def zrow(r, _):
        degl[r, :] = zero16
        return 0
    lax.fori_loop(0, NP // 16, zrow, 0)

    pltpu.sync_copy(col_hbm.at[wid], colv)

    def ebody(j, _):
        for e in range(CH // 16):
            v = colv[j, pl.ds(e * 16, 16)]
            plsc.addupdate_scatter(degl, [v >> 4, v & 15], one16)
        return 0
    lax.fori_loop(0, NCH, ebody, 0)

    pltpu.sync_copy(degl, out_hbm.at[wid])


def _tcd_body(dp_ref, o_ref):
    o_ref[...] = jnp.sum(dp_ref[...], axis=0)


def _tc_degree_reduce(degp):
    return pl.pallas_call(
        _tcd_body,
        out_shape=jax.ShapeDtypeStruct((NP // 128, 128), jnp.float32),
    )(degp)


@functools.partial(
    pl.kernel,
    mesh=_MESH,
    out_type=jax.ShapeDtypeStruct((NC, NP, D), jnp.float32),
    scratch_types=[
        pltpu.VMEM((EW,), jnp.int32),
        pltpu.VMEM((NCH, CH), jnp.int32),
        pltpu.VMEM((CH, D), jnp.float32),
        pltpu.VMEM((CH, D), jnp.float32),
        pltpu.VMEM_SHARED((NP, D), jnp.float32),
        pltpu.SemaphoreType.DMA,
        pltpu.SemaphoreType.DMA,
        pltpu.SemaphoreType.DMA,
        pltpu.SemaphoreType.DMA,
    ],
)
def _sc_conv(g_hbm, rowf_hbm, col_hbm, out_hbm, rowv, colv, rbuf0, rbuf1,
             sacc, gs0, gs1, ss0, ss1):
    c = lax.axis_index("c")
    s = lax.axis_index("s")
    wid = _wid()
    zero16 = jnp.zeros((16,), jnp.float32)

    def zrow(r, _):
        for j in range(D // 16):
            rbuf0[r, pl.ds(j * 16, 16)] = zero16
        return 0
    lax.fori_loop(0, CH, zrow, 0)
    for t in range(RP // CH):
        pltpu.sync_copy(rbuf0, sacc.at[pl.ds(s * RP + t * CH, CH)])

    pltpu.sync_copy(rowf_hbm.at[wid], rowv)
    pltpu.sync_copy(col_hbm.at[wid], colv)
    plsc.subcore_barrier()

    def ridx(j):
        return rowv.at[pl.ds(j * CH, CH)]

    pltpu.async_copy(g_hbm.at[ridx(0)], rbuf0, gs0)

    def body(i, _):
        j0 = 2 * i
        pltpu.make_async_copy(g_hbm.at[ridx(j0)], rbuf0, gs0).wait()

        @pl.when(i > 0)
        def _():
            pltpu.make_async_copy(
                rbuf1, sacc.at[colv.at[j0 - 1]], ss1).wait()
        pltpu.async_copy(g_hbm.at[ridx(j0 + 1)], rbuf1, gs1)
        pltpu.async_copy(rbuf0, sacc.at[colv.at[j0]], ss0, add=True)
        pltpu.make_async_copy(g_hbm.at[ridx(j0 + 1)], rbuf1, gs1).wait()
        pltpu.make_async_copy(rbuf0, sacc.at[colv.at[j0]], ss0).wait()
        pltpu.async_copy(g_hbm.at[ridx(j0 + 2)], rbuf0, gs0)
        pltpu.async_copy(rbuf1, sacc.at[colv.at[j0 + 1]], ss1, add=True)
        return 0
    lax.fori_loop(0, (NCH - 1) // 2, body, 0)

    last = NCH - 1
    pltpu.make_async_copy(g_hbm.at[ridx(last)], rbuf0, gs0).wait()
    pltpu.make_async_copy(rbuf1, sacc.at[colv.at[last - 1]], ss1).wait()
    pltpu.sync_copy(rbuf0, sacc.at[colv.at[last]], add=True)

    plsc.subcore_barrier()
    pltpu.sync_copy(sacc.at[pl.ds(s * RP, RP)],
                    out_hbm.at[c, pl.ds(s * RP, RP)])


EWP = 10240
CHE = 128
NCHE = EWP // CHE


@functools.partial(
    pl.kernel,
    mesh=_MESH,
    out_type=jax.ShapeDtypeStruct((NW, EWP), jnp.float32),
    compiler_params=pltpu.CompilerParams(needs_layout_passes=False),
    scratch_types=[
        pltpu.VMEM((NCHE, CHE), jnp.int32),
        pltpu.VMEM((NCHE, CHE), jnp.int32),
        pltpu.VMEM((CHE, D), jnp.float32),
        pltpu.VMEM((CHE, D), jnp.float32),
        pltpu.VMEM((CHE, D), jnp.float32),
        pltpu.VMEM((CHE, D), jnp.float32),
        pltpu.VMEM((D,), jnp.float32),
        pltpu.VMEM((16,), jnp.float32),
        pltpu.VMEM((CHE,), jnp.float32),
        pltpu.SemaphoreType.DMA,
        pltpu.SemaphoreType.DMA,
        pltpu.SemaphoreType.DMA,
        pltpu.SemaphoreType.DMA,
    ],
)
def _sc_edge(a_hbm, b_hbm, row_hbm, col_hbm, wc2_hbm, bc2_hbm, out_hbm,
             rowv, colv, a0, b0, a1, b1, wv, bcv, obuf,
             ga0, gb0, ga1, gb1):
    wid = _wid()
    pltpu.sync_copy(row_hbm.at[wid], rowv)
    pltpu.sync_copy(col_hbm.at[wid], colv)
    pltpu.sync_copy(wc2_hbm, wv)
    pltpu.sync_copy(bc2_hbm, bcv)

    iota16 = lax.iota(jnp.int32, 16)

    def compute(j, abuf, bbuf):
        bc2 = bcv[...]

        def grp(q, _):
            res = jnp.zeros((16,), jnp.float32)
            for e in range(16):
                r = q * 16 + e
                acc = jnp.zeros((16,), jnp.float32)
                for k in range(D // 16):
                    va = abuf[r, pl.ds(k * 16, 16)]
                    vb = bbuf[r, pl.ds(k * 16, 16)]
                    w = wv[pl.ds(k * 16, 16)]
                    acc = acc + jnp.maximum(va + vb, 0.0) * w
                res = jnp.where(iota16 == e, jnp.sum(acc), res)
            obuf[pl.ds(q * 16, 16)] = 1.0 / (1.0 + jnp.exp(-(res + bc2)))
            return 0
        lax.fori_loop(0, CHE // 16, grp, 0)
        pltpu.sync_copy(obuf, out_hbm.at[wid, pl.ds(j * CHE, CHE)])

    def gathers(j, abuf, bbuf, sa, sb):
        pltpu.async_copy(a_hbm.at[rowv.at[j]], abuf, sa)
        pltpu.async_copy(b_hbm.at[colv.at[j]], bbuf, sb)

    def wait_gathers(j, abuf, bbuf, sa, sb):
        pltpu.make_async_copy(a_hbm.at[rowv.at[j]], abuf, sa).wait()
        pltpu.make_async_copy(b_hbm.at[colv.at[j]], bbuf, sb).wait()

    gathers(0, a0, b0, ga0, gb0)

    def body(i, _):
        j0 = 2 * i
        wait_gathers(j0, a0, b0, ga0, gb0)
        gathers(j0 + 1, a1, b1, ga1, gb1)
        compute(j0, a0, b0)
        wait_gathers(j0 + 1, a1, b1, ga1, gb1)
        gathers(jnp.minimum(j0 + 2, NCHE - 1), a0, b0, ga0, gb0)
        compute(j0 + 1, a1, b1)
        return 0
    lax.fori_loop(0, NCHE // 2, body, 0)

    wait_gathers(NCHE - 1, a0, b0, ga0, gb0)


BM = 1024


def _tc1_body(x_ref, w_ref, d_ref, g_ref, dis_ref):
    deg = d_ref[...] + 1.0
    dis = lax.rsqrt(deg)
    xw = jnp.dot(x_ref[...], w_ref[...], preferred_element_type=jnp.float32)
    g_ref[...] = xw * dis
    dis_ref[...] = dis


def _tc1(xp, W1, dc):
    return pl.pallas_call(
        _tc1_body,
        grid=(NP // BM,),
        in_specs=[
            pl.BlockSpec((BM, D), lambda i: (i, 0)),
            pl.BlockSpec((D, D), lambda i: (0, 0)),
            pl.BlockSpec((BM, 1), lambda i: (i, 0)),
        ],
        out_specs=[
            pl.BlockSpec((BM, D), lambda i: (i, 0)),
            pl.BlockSpec((BM, 1), lambda i: (i, 0)),
        ],
        out_shape=[
            jax.ShapeDtypeStruct((NP, D), jnp.float32),
            jax.ShapeDtypeStruct((NP, 1), jnp.float32),
        ],
    )(xp, W1, dc)


def _tc2_body(p0_ref, p1_ref, g_ref, dis_ref, b_ref, w_ref, o_ref):
    dis = dis_ref[...]
    h = jnp.maximum(dis * (p0_ref[...] + p1_ref[...] + g_ref[...])
                    + b_ref[...], 0.0)
    o_ref[...] = jnp.dot(h, w_ref[...],
                         preferred_element_type=jnp.float32) * dis


def _tc2(p0, p1, g1, dis, b1, W2):
    return pl.pallas_call(
        _tc2_body,
        grid=(NP // BM,),
        in_specs=[
            pl.BlockSpec((BM, D), lambda i: (i, 0)),
            pl.BlockSpec((BM, D), lambda i: (i, 0)),
            pl.BlockSpec((BM, D), lambda i: (i, 0)),
            pl.BlockSpec((BM, 1), lambda i: (i, 0)),
            pl.BlockSpec((1, D), lambda i: (0, 0)),
            pl.BlockSpec((D, D), lambda i: (0, 0)),
        ],
        out_specs=pl.BlockSpec((BM, D), lambda i: (i, 0)),
        out_shape=jax.ShapeDtypeStruct((NP, D), jnp.float32),
    )(p0, p1, g1, dis, b1, W2)


def _tc3_body(q0_ref, q1_ref, g_ref, dis_ref, b_ref, wc1_ref, bc1_ref,
              a_ref, bb_ref):
    dis = dis_ref[...]
    h2 = jnp.maximum(dis * (q0_ref[...] + q1_ref[...] + g_ref[...])
                     + b_ref[...], 0.0)
    a_ref[...] = jnp.dot(h2, wc1_ref[0:D, :],
                         preferred_element_type=jnp.float32) + bc1_ref[...]
    bb_ref[...] = jnp.dot(h2, wc1_ref[D:2 * D, :],
                          preferred_element_type=jnp.float32)


def _tc3(q0, q1, g2, dis, b2, Wc1, bc1):
    return pl.pallas_call(
        _tc3_body,
        grid=(NP // BM,),
        in_specs=[
            pl.BlockSpec((BM, D), lambda i: (i, 0)),
            pl.BlockSpec((BM, D), lambda i: (i, 0)),
            pl.BlockSpec((BM, D), lambda i: (i, 0)),
            pl.BlockSpec((BM, 1), lambda i: (i, 0)),
            pl.BlockSpec((1, D), lambda i: (0, 0)),
            pl.BlockSpec((2 * D, D), lambda i: (0, 0)),
            pl.BlockSpec((1, D), lambda i: (0, 0)),
        ],
        out_specs=[
            pl.BlockSpec((BM, D), lambda i: (i, 0)),
            pl.BlockSpec((BM, D), lambda i: (i, 0)),
        ],
        out_shape=[
            jax.ShapeDtypeStruct((NP, D), jnp.float32),
            jax.ShapeDtypeStruct((NP, D), jnp.float32),
        ],
    )(q0, q1, g2, dis, b2, Wc1, bc1)


def kernel(x, edge_index, W1, b1, W2, b2, Wc1, bc1, Wc2, bc2):
    ei = edge_index.astype(jnp.int32)
    rowf = ei[0].reshape(NW, EW)
    col = ei[1].reshape(NW, NCH, CH)
    pad = jnp.zeros((NW, EWP - EW), jnp.int32)
    rowp = jnp.concatenate([rowf, pad], axis=1).reshape(NW, NCHE, CHE)
    colp = jnp.concatenate([ei[1].reshape(NW, EW), pad],
                           axis=1).reshape(NW, NCHE, CHE)
    xp = jnp.concatenate(
        [x, jnp.zeros((NP - N, D), jnp.float32)], axis=0)

    degp = _sc_degree_local(col).reshape(NW, NP // 128, 128)
    dc = _tc_degree_reduce(degp).reshape(NP, 1)

    g1, dis = _tc1(xp, W1, dc)
    p = _sc_conv(g1, rowf, col)
    g2 = _tc2(p[0], p[1], g1, dis, b1.reshape(1, D), W2)
    q = _sc_conv(g2, rowf, col)
    a, b = _tc3(q[0], q[1], g2, dis, b2.reshape(1, D), Wc1,
                bc1.reshape(1, D))

    wc2v = Wc2.reshape(D)
    bc2v = jnp.broadcast_to(bc2.reshape(1), (16,))
    outp = _sc_edge(a, b, rowp, colp, wc2v, bc2v)
    return outp[:, :EW].reshape(E)


# speedup vs baseline: 1.6832x; 1.6832x over previous
import functools

import jax
import jax.numpy as jnp
from jax import lax
from jax.experimental import pallas as pl
from jax.experimental.pallas import tpu as pltpu
from jax.experimental.pallas import tpu_sc as plsc

N = 10000
D = 128
E = 320000

NC = 2
NS = 16
NW = NC * NS
EW = E // NW
CH = 80
NCH = EW // CH
NP = 10240
RP = NP // NS

_MESH = plsc.VectorSubcoreMesh(core_axis_name="c", subcore_axis_name="s")


def _wid():
    return lax.axis_index("s") * NC + lax.axis_index("c")


@functools.partial(
    pl.kernel,
    mesh=_MESH,
    out_type=jax.ShapeDtypeStruct((NW, NP // 16, 16), jnp.float32),
    compiler_params=pltpu.CompilerParams(needs_layout_passes=False),
    scratch_types=[
        pltpu.VMEM((NCH, CH), jnp.int32),
        pltpu.VMEM((NP // 16, 16), jnp.float32),
    ],
)
def _sc_degree_local(col_hbm, out_hbm, colv, degl):
    wid = _wid()
    zero16 = jnp.zeros((16,), jnp.float32)
    one16 = jnp.ones((16,), jnp.float32)

    def zrow(r, _):
        degl[r, :] = zero16
        return 0
    lax.fori_loop(0, NP // 16, zrow, 0)

    pltpu.sync_copy(col_hbm.at[wid], colv)

    def ebody(j, _):
        for e in range(CH // 16):
            v = colv[j, pl.ds(e * 16, 16)]
            plsc.addupdate_scatter(degl, [v >> 4, v & 15], one16)
        return 0
    lax.fori_loop(0, NCH, ebody, 0)

    pltpu.sync_copy(degl, out_hbm.at[wid])


def _tcd_body(dp_ref, o_ref):
    o_ref[...] = jnp.sum(dp_ref[...], axis=0)


def _tc_degree_reduce(degp):
    return pl.pallas_call(
        _tcd_body,
        out_shape=jax.ShapeDtypeStruct((NP // 128, 128), jnp.float32),
    )(degp)


@functools.partial(
    pl.kernel,
    mesh=_MESH,
    out_type=jax.ShapeDtypeStruct((NC, NP, D), jnp.float32),
    scratch_types=[
        pltpu.VMEM((EW,), jnp.int32),
        pltpu.VMEM((NCH, CH), jnp.int32),
        pltpu.VMEM((CH, D), jnp.float32),
        pltpu.VMEM((CH, D), jnp.float32),
        pltpu.VMEM_SHARED((NP, D), jnp.float32),
        pltpu.SemaphoreType.DMA,
        pltpu.SemaphoreType.DMA,
        pltpu.SemaphoreType.DMA,
        pltpu.SemaphoreType.DMA,
    ],
)
def _sc_conv(g_hbm, rowf_hbm, col_hbm, out_hbm, rowv, colv, rbuf0, rbuf1,
             sacc, gs0, gs1, ss0, ss1):
    c = lax.axis_index("c")
    s = lax.axis_index("s")
    wid = _wid()
    zero16 = jnp.zeros((16,), jnp.float32)

    def zrow(r, _):
        for j in range(D // 16):
            rbuf0[r, pl.ds(j * 16, 16)] = zero16
        return 0
    lax.fori_loop(0, CH, zrow, 0)
    for t in range(RP // CH):
        pltpu.sync_copy(rbuf0, sacc.at[pl.ds(s * RP + t * CH, CH)])

    pltpu.sync_copy(rowf_hbm.at[wid], rowv)
    pltpu.sync_copy(col_hbm.at[wid], colv)
    plsc.subcore_barrier()

    def ridx(j):
        return rowv.at[pl.ds(j * CH, CH)]

    pltpu.async_copy(g_hbm.at[ridx(0)], rbuf0, gs0)

    def body(i, _):
        j0 = 2 * i
        pltpu.make_async_copy(g_hbm.at[ridx(j0)], rbuf0, gs0).wait()

        @pl.when(i > 0)
        def _():
            pltpu.make_async_copy(
                rbuf1, sacc.at[colv.at[j0 - 1]], ss1).wait()
        pltpu.async_copy(g_hbm.at[ridx(j0 + 1)], rbuf1, gs1)
        pltpu.async_copy(rbuf0, sacc.at[colv.at[j0]], ss0, add=True)
        pltpu.make_async_copy(g_hbm.at[ridx(j0 + 1)], rbuf1, gs1).wait()
        pltpu.make_async_copy(rbuf0, sacc.at[colv.at[j0]], ss0).wait()
        pltpu.async_copy(g_hbm.at[ridx(j0 + 2)], rbuf0, gs0)
        pltpu.async_copy(rbuf1, sacc.at[colv.at[j0 + 1]], ss1, add=True)
        return 0
    lax.fori_loop(0, (NCH - 1) // 2, body, 0)

    last = NCH - 1
    pltpu.make_async_copy(g_hbm.at[ridx(last)], rbuf0, gs0).wait()
    pltpu.make_async_copy(rbuf1, sacc.at[colv.at[last - 1]], ss1).wait()
    pltpu.sync_copy(rbuf0, sacc.at[colv.at[last]], add=True)

    plsc.subcore_barrier()
    pltpu.sync_copy(sacc.at[pl.ds(s * RP, RP)],
                    out_hbm.at[c, pl.ds(s * RP, RP)])


CHE = CH
NCHE = NCH


@functools.partial(
    pl.kernel,
    mesh=_MESH,
    out_type=jax.ShapeDtypeStruct((E,), jnp.float32),
    compiler_params=pltpu.CompilerParams(needs_layout_passes=False),
    scratch_types=[
        pltpu.VMEM((NCHE, CHE), jnp.int32),
        pltpu.VMEM((NCHE, CHE), jnp.int32),
        pltpu.VMEM((CHE, D), jnp.float32),
        pltpu.VMEM((CHE, D), jnp.float32),
        pltpu.VMEM((CHE, D), jnp.float32),
        pltpu.VMEM((CHE, D), jnp.float32),
        pltpu.VMEM((D,), jnp.float32),
        pltpu.VMEM((16,), jnp.float32),
        pltpu.VMEM((CHE,), jnp.float32),
        pltpu.SemaphoreType.DMA,
        pltpu.SemaphoreType.DMA,
        pltpu.SemaphoreType.DMA,
        pltpu.SemaphoreType.DMA,
    ],
)
def _sc_edge(a_hbm, b_hbm, row_hbm, col_hbm, wc2_hbm, bc2_hbm, out_hbm,
             rowv, colv, a0, b0, a1, b1, wv, bcv, obuf,
             ga0, gb0, ga1, gb1):
    wid = _wid()
    pltpu.sync_copy(row_hbm.at[wid], rowv)
    pltpu.sync_copy(col_hbm.at[wid], colv)
    pltpu.sync_copy(wc2_hbm, wv)
    pltpu.sync_copy(bc2_hbm, bcv)

    iota16 = lax.iota(jnp.int32, 16)

    def compute(j, abuf, bbuf):
        bc2 = bcv[...]

        def grp(q, _):
            res = jnp.zeros((16,), jnp.float32)
            for e in range(16):
                r = q * 16 + e
                acc = jnp.zeros((16,), jnp.float32)
                for k in range(D // 16):
                    va = abuf[r, pl.ds(k * 16, 16)]
                    vb = bbuf[r, pl.ds(k * 16, 16)]
                    w = wv[pl.ds(k * 16, 16)]
                    acc = acc + jnp.maximum(va + vb, 0.0) * w
                res = jnp.where(iota16 == e, jnp.sum(acc), res)
            obuf[pl.ds(q * 16, 16)] = 1.0 / (1.0 + jnp.exp(-(res + bc2)))
            return 0
        lax.fori_loop(0, CHE // 16, grp, 0)
        pltpu.sync_copy(obuf, out_hbm.at[pl.ds(wid * EW + j * CHE, CHE)])

    def gathers(j, abuf, bbuf, sa, sb):
        pltpu.async_copy(a_hbm.at[rowv.at[j]], abuf, sa)
        pltpu.async_copy(b_hbm.at[colv.at[j]], bbuf, sb)

    def wait_gathers(j, abuf, bbuf, sa, sb):
        pltpu.make_async_copy(a_hbm.at[rowv.at[j]], abuf, sa).wait()
        pltpu.make_async_copy(b_hbm.at[colv.at[j]], bbuf, sb).wait()

    gathers(0, a0, b0, ga0, gb0)

    def body(i, _):
        j0 = 2 * i
        wait_gathers(j0, a0, b0, ga0, gb0)
        gathers(j0 + 1, a1, b1, ga1, gb1)
        compute(j0, a0, b0)
        wait_gathers(j0 + 1, a1, b1, ga1, gb1)
        gathers(jnp.minimum(j0 + 2, NCHE - 1), a0, b0, ga0, gb0)
        compute(j0 + 1, a1, b1)
        return 0
    lax.fori_loop(0, NCHE // 2, body, 0)

    wait_gathers(NCHE - 1, a0, b0, ga0, gb0)
    compute(NCHE - 1, a0, b0)


BM = 1024


def _tc1_body(x_ref, w_ref, d_ref, g_ref, dis_ref):
    deg = d_ref[...] + 1.0
    dis = lax.rsqrt(deg)
    xw = jnp.dot(x_ref[...], w_ref[...], preferred_element_type=jnp.float32)
    g_ref[...] = xw * dis
    dis_ref[...] = dis


def _tc1(xp, W1, dc):
    return pl.pallas_call(
        _tc1_body,
        grid=(NP // BM,),
        in_specs=[
            pl.BlockSpec((BM, D), lambda i: (i, 0)),
            pl.BlockSpec((D, D), lambda i: (0, 0)),
            pl.BlockSpec((BM, 1), lambda i: (i, 0)),
        ],
        out_specs=[
            pl.BlockSpec((BM, D), lambda i: (i, 0)),
            pl.BlockSpec((BM, 1), lambda i: (i, 0)),
        ],
        out_shape=[
            jax.ShapeDtypeStruct((NP, D), jnp.float32),
            jax.ShapeDtypeStruct((NP, 1), jnp.float32),
        ],
    )(xp, W1, dc)


def _tc2_body(p0_ref, p1_ref, g_ref, dis_ref, b_ref, w_ref, o_ref):
    dis = dis_ref[...]
    h = jnp.maximum(dis * (p0_ref[...] + p1_ref[...] + g_ref[...])
                    + b_ref[...], 0.0)
    o_ref[...] = jnp.dot(h, w_ref[...],
                         preferred_element_type=jnp.float32) * dis


def _tc2(p0, p1, g1, dis, b1, W2):
    return pl.pallas_call(
        _tc2_body,
        grid=(NP // BM,),
        in_specs=[
            pl.BlockSpec((BM, D), lambda i: (i, 0)),
            pl.BlockSpec((BM, D), lambda i: (i, 0)),
            pl.BlockSpec((BM, D), lambda i: (i, 0)),
            pl.BlockSpec((BM, 1), lambda i: (i, 0)),
            pl.BlockSpec((1, D), lambda i: (0, 0)),
            pl.BlockSpec((D, D), lambda i: (0, 0)),
        ],
        out_specs=pl.BlockSpec((BM, D), lambda i: (i, 0)),
        out_shape=jax.ShapeDtypeStruct((NP, D), jnp.float32),
    )(p0, p1, g1, dis, b1, W2)


def _tc3_body(q0_ref, q1_ref, g_ref, dis_ref, b_ref, wc1_ref, bc1_ref,
              a_ref, bb_ref):
    dis = dis_ref[...]
    h2 = jnp.maximum(dis * (q0_ref[...] + q1_ref[...] + g_ref[...])
                     + b_ref[...], 0.0)
    a_ref[...] = jnp.dot(h2, wc1_ref[0:D, :],
                         preferred_element_type=jnp.float32) + bc1_ref[...]
    bb_ref[...] = jnp.dot(h2, wc1_ref[D:2 * D, :],
                          preferred_element_type=jnp.float32)


def _tc3(q0, q1, g2, dis, b2, Wc1, bc1):
    return pl.pallas_call(
        _tc3_body,
        grid=(NP // BM,),
        in_specs=[
            pl.BlockSpec((BM, D), lambda i: (i, 0)),
            pl.BlockSpec((BM, D), lambda i: (i, 0)),
            pl.BlockSpec((BM, D), lambda i: (i, 0)),
            pl.BlockSpec((BM, 1), lambda i: (i, 0)),
            pl.BlockSpec((1, D), lambda i: (0, 0)),
            pl.BlockSpec((2 * D, D), lambda i: (0, 0)),
            pl.BlockSpec((1, D), lambda i: (0, 0)),
        ],
        out_specs=[
            pl.BlockSpec((BM, D), lambda i: (i, 0)),
            pl.BlockSpec((BM, D), lambda i: (i, 0)),
        ],
        out_shape=[
            jax.ShapeDtypeStruct((NP, D), jnp.float32),
            jax.ShapeDtypeStruct((NP, D), jnp.float32),
        ],
    )(q0, q1, g2, dis, b2, Wc1, bc1)


def kernel(x, edge_index, W1, b1, W2, b2, Wc1, bc1, Wc2, bc2):
    ei = edge_index.astype(jnp.int32)
    rowf = ei[0].reshape(NW, EW)
    col = ei[1].reshape(NW, NCH, CH)
    rowp = rowf.reshape(NW, NCHE, CHE)
    colp = ei[1].reshape(NW, NCHE, CHE)
    xp = jnp.concatenate(
        [x, jnp.zeros((NP - N, D), jnp.float32)], axis=0)

    degp = _sc_degree_local(col).reshape(NW, NP // 128, 128)
    dc = _tc_degree_reduce(degp).reshape(NP, 1)

    g1, dis = _tc1(xp, W1, dc)
    p = _sc_conv(g1, rowf, col)
    g2 = _tc2(p[0], p[1], g1, dis, b1.reshape(1, D), W2)
    q = _sc_conv(g2, rowf, col)
    a, b = _tc3(q[0], q[1], g2, dis, b2.reshape(1, D), Wc1,
                bc1.reshape(1, D))

    wc2v = Wc2.reshape(D)
    bc2v = jnp.broadcast_to(bc2.reshape(1), (16,))
    return _sc_edge(a, b, rowp, colp, wc2v, bc2v)


# speedup vs baseline: 1.6877x; 1.0026x over previous
import functools

import jax
import jax.numpy as jnp
from jax import lax
from jax.experimental import pallas as pl
from jax.experimental.pallas import tpu as pltpu
from jax.experimental.pallas import tpu_sc as plsc

N = 10000
D = 128
E = 320000

NC = 2
NS = 16
NW = NC * NS
EW = E // NW
CH = 80
NCH = EW // CH
NP = 10240
RP = NP // NS

_MESH = plsc.VectorSubcoreMesh(core_axis_name="c", subcore_axis_name="s")


def _wid():
    return lax.axis_index("s") * NC + lax.axis_index("c")


@functools.partial(
    pl.kernel,
    mesh=_MESH,
    out_type=jax.ShapeDtypeStruct((NW, NP // 16, 16), jnp.float32),
    compiler_params=pltpu.CompilerParams(needs_layout_passes=False),
    scratch_types=[
        pltpu.VMEM((NCH, CH), jnp.int32),
        pltpu.VMEM((NP // 16, 16), jnp.float32),
    ],
)
def _sc_degree_local(col_hbm, out_hbm, colv, degl):
    wid = _wid()
    zero16 = jnp.zeros((16,), jnp.float32)
    one16 = jnp.ones((16,), jnp.float32)

    def zrow(r, _):
        degl[r, :] = zero16
        return 0
    lax.fori_loop(0, NP // 16, zrow, 0)

    pltpu.sync_copy(col_hbm.at[wid], colv)

    def ebody(j, _):
        for e in range(CH // 16):
            v = colv[j, pl.ds(e * 16, 16)]
            plsc.addupdate_scatter(degl, [v >> 4, v & 15], one16)
        return 0
    lax.fori_loop(0, NCH, ebody, 0)

    pltpu.sync_copy(degl, out_hbm.at[wid])


def _tcd_body(dp_ref, o_ref):
    o_ref[...] = jnp.sum(dp_ref[...], axis=0)


def _tc_degree_reduce(degp):
    return pl.pallas_call(
        _tcd_body,
        out_shape=jax.ShapeDtypeStruct((NP // 128, 128), jnp.float32),
    )(degp)


@functools.partial(
    pl.kernel,
    mesh=_MESH,
    out_type=jax.ShapeDtypeStruct((NC, NP, D), jnp.float32),
    scratch_types=[
        pltpu.VMEM((EW,), jnp.int32),
        pltpu.VMEM((NCH, CH), jnp.int32),
        pltpu.VMEM((CH, D), jnp.float32),
        pltpu.VMEM((CH, D), jnp.float32),
        pltpu.VMEM_SHARED((NP, D), jnp.float32),
        pltpu.SemaphoreType.DMA,
        pltpu.SemaphoreType.DMA,
        pltpu.SemaphoreType.DMA,
        pltpu.SemaphoreType.DMA,
    ],
)
def _sc_conv(g_hbm, rowf_hbm, col_hbm, out_hbm, rowv, colv, rbuf0, rbuf1,
             sacc, gs0, gs1, ss0, ss1):
    c = lax.axis_index("c")
    s = lax.axis_index("s")
    wid = _wid()
    zero16 = jnp.zeros((16,), jnp.float32)

    def zrow(r, _):
        for j in range(D // 16):
            rbuf0[r, pl.ds(j * 16, 16)] = zero16
        return 0
    lax.fori_loop(0, CH, zrow, 0)
    for t in range(RP // CH):
        pltpu.sync_copy(rbuf0, sacc.at[pl.ds(s * RP + t * CH, CH)])

    pltpu.sync_copy(rowf_hbm.at[wid], rowv)
    pltpu.sync_copy(col_hbm.at[wid], colv)
    plsc.subcore_barrier()

    def ridx(j):
        return rowv.at[pl.ds(j * CH, CH)]

    pltpu.async_copy(g_hbm.at[ridx(0)], rbuf0, gs0)

    def body(i, _):
        j0 = 2 * i
        pltpu.make_async_copy(g_hbm.at[ridx(j0)], rbuf0, gs0).wait()

        @pl.when(i > 0)
        def _():
            pltpu.make_async_copy(
                rbuf1, sacc.at[colv.at[j0 - 1]], ss1).wait()
        pltpu.async_copy(g_hbm.at[ridx(j0 + 1)], rbuf1, gs1)
        pltpu.async_copy(rbuf0, sacc.at[colv.at[j0]], ss0, add=True)
        pltpu.make_async_copy(g_hbm.at[ridx(j0 + 1)], rbuf1, gs1).wait()
        pltpu.make_async_copy(rbuf0, sacc.at[colv.at[j0]], ss0).wait()
        pltpu.async_copy(g_hbm.at[ridx(j0 + 2)], rbuf0, gs0)
        pltpu.async_copy(rbuf1, sacc.at[colv.at[j0 + 1]], ss1, add=True)
        return 0
    lax.fori_loop(0, (NCH - 1) // 2, body, 0)

    last = NCH - 1
    pltpu.make_async_copy(g_hbm.at[ridx(last)], rbuf0, gs0).wait()
    pltpu.make_async_copy(rbuf1, sacc.at[colv.at[last - 1]], ss1).wait()
    pltpu.sync_copy(rbuf0, sacc.at[colv.at[last]], add=True)

    plsc.subcore_barrier()
    pltpu.sync_copy(sacc.at[pl.ds(s * RP, RP)],
                    out_hbm.at[c, pl.ds(s * RP, RP)])


CHE = CH
NCHE = NCH


@functools.partial(
    pl.kernel,
    mesh=_MESH,
    out_type=jax.ShapeDtypeStruct((E,), jnp.float32),
    compiler_params=pltpu.CompilerParams(needs_layout_passes=False),
    scratch_types=[
        pltpu.VMEM((NCHE, CHE), jnp.int32),
        pltpu.VMEM((NCHE, CHE), jnp.int32),
        pltpu.VMEM((CHE, D), jnp.float32),
        pltpu.VMEM((CHE, D), jnp.float32),
        pltpu.VMEM((CHE, D), jnp.float32),
        pltpu.VMEM((CHE, D), jnp.float32),
        pltpu.VMEM((D,), jnp.float32),
        pltpu.VMEM((16,), jnp.float32),
        pltpu.VMEM((CHE,), jnp.float32),
        pltpu.VMEM((CHE,), jnp.float32),
        pltpu.SemaphoreType.DMA,
        pltpu.SemaphoreType.DMA,
        pltpu.SemaphoreType.DMA,
        pltpu.SemaphoreType.DMA,
        pltpu.SemaphoreType.DMA,
        pltpu.SemaphoreType.DMA,
    ],
)
def _sc_edge(a_hbm, b_hbm, row_hbm, col_hbm, wc2_hbm, bc2_hbm, out_hbm,
             rowv, colv, a0, b0, a1, b1, wv, bcv, o0, o1,
             ga0, gb0, ga1, gb1, so0, so1):
    wid = _wid()
    pltpu.sync_copy(row_hbm.at[wid], rowv)
    pltpu.sync_copy(col_hbm.at[wid], colv)
    pltpu.sync_copy(wc2_hbm, wv)
    pltpu.sync_copy(bc2_hbm, bcv)

    iota16 = lax.iota(jnp.int32, 16)

    def compute(j, abuf, bbuf, obuf, so):
        bc2 = bcv[...]

        def grp(q, _):
            res = jnp.zeros((16,), jnp.float32)
            for e in range(16):
                r = q * 16 + e
                acc = jnp.zeros((16,), jnp.float32)
                for k in range(D // 16):
                    va = abuf[r, pl.ds(k * 16, 16)]
                    vb = bbuf[r, pl.ds(k * 16, 16)]
                    w = wv[pl.ds(k * 16, 16)]
                    acc = acc + jnp.maximum(va + vb, 0.0) * w
                res = jnp.where(iota16 == e, jnp.sum(acc), res)
            obuf[pl.ds(q * 16, 16)] = 1.0 / (1.0 + jnp.exp(-(res + bc2)))
            return 0
        lax.fori_loop(0, CHE // 16, grp, 0)
        pltpu.async_copy(obuf, out_hbm.at[pl.ds(wid * EW + j * CHE, CHE)],
                         so)

    def wait_out(j, obuf, so):
        pltpu.make_async_copy(
            obuf, out_hbm.at[pl.ds(wid * EW + j * CHE, CHE)], so).wait()

    def gathers(j, abuf, bbuf, sa, sb):
        pltpu.async_copy(a_hbm.at[rowv.at[j]], abuf, sa)
        pltpu.async_copy(b_hbm.at[colv.at[j]], bbuf, sb)

    def wait_gathers(j, abuf, bbuf, sa, sb):
        pltpu.make_async_copy(a_hbm.at[rowv.at[j]], abuf, sa).wait()
        pltpu.make_async_copy(b_hbm.at[colv.at[j]], bbuf, sb).wait()

    gathers(0, a0, b0, ga0, gb0)

    def body(i, _):
        j0 = 2 * i
        wait_gathers(j0, a0, b0, ga0, gb0)
        gathers(j0 + 1, a1, b1, ga1, gb1)

        @pl.when(i > 0)
        def _():
            wait_out(j0 - 2, o0, so0)
        compute(j0, a0, b0, o0, so0)
        wait_gathers(j0 + 1, a1, b1, ga1, gb1)
        gathers(jnp.minimum(j0 + 2, NCHE - 1), a0, b0, ga0, gb0)

        @pl.when(i > 0)
        def _():
            wait_out(j0 - 1, o1, so1)
        compute(j0 + 1, a1, b1, o1, so1)
        return 0
    lax.fori_loop(0, NCHE // 2, body, 0)

    last = NCHE - 1
    wait_gathers(last, a0, b0, ga0, gb0)
    wait_out(last - 3, o0, so0)
    compute(last, a0, b0, o0, so0)
    wait_out(last - 2, o1, so1)
    wait_out(last, o0, so0)


BM = 1024


def _tc1_body(x_ref, w_ref, d_ref, g_ref, dis_ref):
    deg = d_ref[...] + 1.0
    dis = lax.rsqrt(deg)
    xw = jnp.dot(x_ref[...], w_ref[...], preferred_element_type=jnp.float32)
    g_ref[...] = xw * dis
    dis_ref[...] = dis


def _tc1(xp, W1, dc):
    return pl.pallas_call(
        _tc1_body,
        grid=(NP // BM,),
        in_specs=[
            pl.BlockSpec((BM, D), lambda i: (i, 0)),
            pl.BlockSpec((D, D), lambda i: (0, 0)),
            pl.BlockSpec((BM, 1), lambda i: (i, 0)),
        ],
        out_specs=[
            pl.BlockSpec((BM, D), lambda i: (i, 0)),
            pl.BlockSpec((BM, 1), lambda i: (i, 0)),
        ],
        out_shape=[
            jax.ShapeDtypeStruct((NP, D), jnp.float32),
            jax.ShapeDtypeStruct((NP, 1), jnp.float32),
        ],
    )(xp, W1, dc)


def _tc2_body(p0_ref, p1_ref, g_ref, dis_ref, b_ref, w_ref, o_ref):
    dis = dis_ref[...]
    h = jnp.maximum(dis * (p0_ref[...] + p1_ref[...] + g_ref[...])
                    + b_ref[...], 0.0)
    o_ref[...] = jnp.dot(h, w_ref[...],
                         preferred_element_type=jnp.float32) * dis


def _tc2(p0, p1, g1, dis, b1, W2):
    return pl.pallas_call(
        _tc2_body,
        grid=(NP // BM,),
        in_specs=[
            pl.BlockSpec((BM, D), lambda i: (i, 0)),
            pl.BlockSpec((BM, D), lambda i: (i, 0)),
            pl.BlockSpec((BM, D), lambda i: (i, 0)),
            pl.BlockSpec((BM, 1), lambda i: (i, 0)),
            pl.BlockSpec((1, D), lambda i: (0, 0)),
            pl.BlockSpec((D, D), lambda i: (0, 0)),
        ],
        out_specs=pl.BlockSpec((BM, D), lambda i: (i, 0)),
        out_shape=jax.ShapeDtypeStruct((NP, D), jnp.float32),
    )(p0, p1, g1, dis, b1, W2)


def _tc3_body(q0_ref, q1_ref, g_ref, dis_ref, b_ref, wc1_ref, bc1_ref,
              a_ref, bb_ref):
    dis = dis_ref[...]
    h2 = jnp.maximum(dis * (q0_ref[...] + q1_ref[...] + g_ref[...])
                     + b_ref[...], 0.0)
    a_ref[...] = jnp.dot(h2, wc1_ref[0:D, :],
                         preferred_element_type=jnp.float32) + bc1_ref[...]
    bb_ref[...] = jnp.dot(h2, wc1_ref[D:2 * D, :],
                          preferred_element_type=jnp.float32)


def _tc3(q0, q1, g2, dis, b2, Wc1, bc1):
    return pl.pallas_call(
        _tc3_body,
        grid=(NP // BM,),
        in_specs=[
            pl.BlockSpec((BM, D), lambda i: (i, 0)),
            pl.BlockSpec((BM, D), lambda i: (i, 0)),
            pl.BlockSpec((BM, D), lambda i: (i, 0)),
            pl.BlockSpec((BM, 1), lambda i: (i, 0)),
            pl.BlockSpec((1, D), lambda i: (0, 0)),
            pl.BlockSpec((2 * D, D), lambda i: (0, 0)),
            pl.BlockSpec((1, D), lambda i: (0, 0)),
        ],
        out_specs=[
            pl.BlockSpec((BM, D), lambda i: (i, 0)),
            pl.BlockSpec((BM, D), lambda i: (i, 0)),
        ],
        out_shape=[
            jax.ShapeDtypeStruct((NP, D), jnp.float32),
            jax.ShapeDtypeStruct((NP, D), jnp.float32),
        ],
    )(q0, q1, g2, dis, b2, Wc1, bc1)


def kernel(x, edge_index, W1, b1, W2, b2, Wc1, bc1, Wc2, bc2):
    ei = edge_index.astype(jnp.int32)
    rowf = ei[0].reshape(NW, EW)
    col = ei[1].reshape(NW, NCH, CH)
    rowp = rowf.reshape(NW, NCHE, CHE)
    colp = ei[1].reshape(NW, NCHE, CHE)
    xp = jnp.concatenate(
        [x, jnp.zeros((NP - N, D), jnp.float32)], axis=0)

    degp = _sc_degree_local(col).reshape(NW, NP // 128, 128)
    dc = _tc_degree_reduce(degp).reshape(NP, 1)

    g1, dis = _tc1(xp, W1, dc)
    p = _sc_conv(g1, rowf, col)
    g2 = _tc2(p[0], p[1], g1, dis, b1.reshape(1, D), W2)
    q = _sc_conv(g2, rowf, col)
    a, b = _tc3(q[0], q[1], g2, dis, b2.reshape(1, D), Wc1,
                bc1.reshape(1, D))

    wc2v = Wc2.reshape(D)
    bc2v = jnp.broadcast_to(bc2.reshape(1), (16,))
    return _sc_edge(a, b, rowp, colp, wc2v, bc2v)
